# baseline (speedup 1.0000x reference)
import functools

import jax
import jax.numpy as jnp
from jax import lax
from jax.experimental import pallas as pl
from jax.experimental.pallas import tpu as pltpu
from jax.experimental.pallas import tpu_sc as plsc

D = 64
NC = 2
NS = 16
BN = 256

CHUNK = 128
GRP = 8
EG = CHUNK * GRP


def _sc_round_body(nchunks_per_tile, ngroups_per_tile, zrows,
                   h_hbm, src_hbm, ldst_hbm, z2_hbm, agg_hbm,
                   acc, sidx, ldstv, rows, gsem0, gsem1, ssem0, ssem1):
  gsems = [gsem0, gsem1]
  ssems = [ssem0, ssem1]
  c = lax.axis_index("c")
  s = lax.axis_index("s")
  for b in range(2):
    pltpu.sync_copy(z2_hbm, acc.at[pl.ds(s * zrows, zrows)])
    plsc.subcore_barrier()

    def grp_body(g, _):
      base = s * nchunks_per_tile + g * GRP
      pltpu.sync_copy(src_hbm.at[pl.ds(base, GRP)], sidx)
      pltpu.sync_copy(ldst_hbm.at[c, pl.ds(base, GRP)], ldstv)
      gds = [None] * GRP
      sds = [None] * GRP
      gds[0] = pltpu.async_copy(h_hbm.at[b].at[sidx.at[0]], rows.at[0],
                                gsems[0])
      for j in range(GRP):
        if j >= 1:
          sds[j - 1].wait()
        if j + 1 < GRP:
          gds[j + 1] = pltpu.async_copy(h_hbm.at[b].at[sidx.at[j + 1]],
                                        rows.at[(j + 1) % 2],
                                        gsems[(j + 1) % 2])
        gds[j].wait()
        sds[j] = pltpu.async_copy(rows.at[j % 2], acc.at[ldstv.at[j]],
                                  ssems[j % 2], add=True)
      sds[GRP - 1].wait()
      return 0

    lax.fori_loop(0, ngroups_per_tile, grp_body, 0)
    plsc.subcore_barrier()
    pltpu.sync_copy(acc.at[pl.ds(s * zrows, zrows)],
                    agg_hbm.at[b, c, pl.ds(s * zrows, zrows)])
    if b == 0:
      plsc.subcore_barrier()


def _sc_deg_body(nchunks_per_tile, ngroups_per_tile, zrows,
                 w_hbm, ldst_hbm, z1_hbm, deg_hbm,
                 acc1, wv, ldstv, zb1):
  c = lax.axis_index("c")
  s = lax.axis_index("s")
  pltpu.sync_copy(z1_hbm, zb1)
  pltpu.sync_copy(zb1, acc1.at[pl.ds(s * zrows, zrows)])
  plsc.subcore_barrier()

  def grp_body(g, _):
    base = s * nchunks_per_tile + g * GRP
    pltpu.sync_copy(w_hbm.at[pl.ds(base, GRP)], wv)
    pltpu.sync_copy(ldst_hbm.at[c, pl.ds(base, GRP)], ldstv)
    for j in range(GRP):
      pltpu.sync_copy(wv.at[j], acc1.at[ldstv.at[j]], add=True)
    return 0

  lax.fori_loop(0, ngroups_per_tile, grp_body, 0)
  plsc.subcore_barrier()
  halloc = zrows * NS
  pltpu.sync_copy(acc1.at[pl.ds(s * zrows, zrows)], zb1)
  pltpu.sync_copy(zb1, deg_hbm.at[pl.ds(c * halloc + s * zrows, zrows)])


def _base_term(g_blk, t1, t4, t3):
  cvec = lax.dot_general(jnp.maximum(t4, 0.0), t3,
                         (((1,), (1,)), ((), ())))
  return g_blk[:, 0:1] * t1 + g_blk[:, 1:2] * cvec


def _tc_round1_body(g_ref, t1_ref, t4_ref, t3_ref, t2_ref, h_ref):
  base = _base_term(g_ref[0], t1_ref[...], t4_ref[...], t3_ref[...])
  mu = jnp.maximum(base, 0.0)
  h_ref[0] = lax.dot_general(mu, t2_ref[...], (((1,), (1,)), ((), ())))


def _tc_round_body(g_ref, agg_ref, t1_ref, t4_ref, t3_ref, t2_ref, h_ref):
  base = _base_term(g_ref[0], t1_ref[...], t4_ref[...], t3_ref[...])
  mu = jnp.maximum(base + agg_ref[0, 0], 0.0)
  h_ref[0] = lax.dot_general(mu, t2_ref[...], (((1,), (1,)), ((), ())))


def _tc_final_round_body(nb_per_core, h_half,
                         g_ref, agg_ref, vs_ref, t1_ref, t4_ref, t3_ref,
                         musum_ref, muv_ref):
  b = pl.program_id(0)
  i = pl.program_id(1)
  base = _base_term(g_ref[0], t1_ref[...], t4_ref[...], t3_ref[...])
  mu = jnp.maximum(base + agg_ref[0, 0], 0.0)

  @pl.when(i == 0)
  def _():
    musum_ref[...] = jnp.zeros_like(musum_ref)
    muv_ref[...] = jnp.zeros_like(muv_ref)

  musum_ref[...] += jnp.sum(mu, axis=0)[None, None, :]
  r = vs_ref[b] - (i // nb_per_core) * h_half - (i % nb_per_core) * BN
  iota = lax.broadcasted_iota(jnp.int32, (BN, 1), 0)
  hit = jnp.where(iota == r, mu, 0.0)
  muv_ref[...] += jnp.sum(hit, axis=0)[None, None, :]


def _tc_q_body(musum_ref, muv_ref, t5_ref, t6_ref, t7_ref, q_ref):
  ms = musum_ref[0]
  mv = muv_ref[0]
  h6 = jnp.maximum(lax.dot_general(ms, t6_ref[...], (((1,), (1,)), ((), ()))), 0.0)
  h7 = jnp.maximum(lax.dot_general(mv, t7_ref[...], (((1,), (1,)), ((), ()))), 0.0)
  t5 = t5_ref[...]
  q = jnp.sum(h6 * t5[:, :D]) + jnp.sum(h7 * t5[:, D:])
  q_ref[...] = jnp.zeros_like(q_ref) + q


@jax.jit
def kernel(xv, vs, edge_index, edge_weight, theta1, theta2, theta3, theta4,
           theta5, theta6, theta7):
  B, N = xv.shape
  E = edge_index.shape[1]

  h_half = ((N + 2 * BN - 1) // (2 * BN)) * BN
  npad = 2 * h_half
  nb_per_core = h_half // BN
  nblk = npad // BN
  h_alloc = ((h_half + BN) // BN) * BN
  ndump = h_alloc - h_half
  zrows = h_alloc // NS
  ept = ((E + NS * EG - 1) // (NS * EG)) * EG
  e_pad = NS * ept
  nchunks_per_tile = ept // CHUNK
  ngroups_per_tile = ept // EG
  npad_edges = e_pad - E

  f32 = jnp.float32

  src = edge_index[0]
  dst = edge_index[1]
  pad_src = jnp.arange(npad_edges, dtype=jnp.int32) % N
  srcp = jnp.concatenate([src, pad_src]).reshape(e_pad // CHUNK, CHUNK)
  dstp = jnp.concatenate(
      [dst, jnp.full((npad_edges,), npad, jnp.int32)])
  lane = jnp.arange(e_pad, dtype=jnp.int32) % ndump
  dump = h_half + lane
  ldst0 = jnp.where(dstp < h_half, dstp, dump)
  ldst1 = jnp.where((dstp >= h_half) & (dstp < npad), dstp - h_half, dump)
  ldst = jnp.stack([ldst0, ldst1]).reshape(NC, e_pad // CHUNK, CHUNK)
  wp = jnp.concatenate([edge_weight, jnp.zeros((npad_edges,), f32)]
                       ).reshape(e_pad // CHUNK, CHUNK)
  xp = jnp.concatenate([xv, jnp.zeros((B, npad - N), f32)], axis=1)
  z1 = jnp.zeros((zrows,), f32)
  z2 = jnp.zeros((zrows, D), f32)
  t1r = theta1.reshape(1, D)
  t4r = theta4.reshape(1, D)
  t5r = theta5.reshape(1, 2 * D)

  mesh = plsc.VectorSubcoreMesh(core_axis_name="c", subcore_axis_name="s",
                                num_cores=NC, num_subcores=NS)

  deg_kernel = pl.kernel(
      functools.partial(_sc_deg_body, nchunks_per_tile, ngroups_per_tile,
                        zrows),
      out_type=jax.ShapeDtypeStruct((NC * h_alloc,), f32),
      mesh=mesh,
      scratch_types=[
          pltpu.VMEM_SHARED((h_alloc,), f32),
          pltpu.VMEM((GRP, CHUNK), f32),
          pltpu.VMEM((GRP, CHUNK), jnp.int32),
          pltpu.VMEM((zrows,), f32),
      ],
  )
  deg_sc = deg_kernel(wp, ldst, z1)
  deg_lin = deg_sc.reshape(NC, h_alloc)[:, :h_half].reshape(npad)

  g = jnp.stack([xp, jnp.broadcast_to(deg_lin[None, :], (B, npad))],
                axis=-1)

  round_kernel = pl.kernel(
      functools.partial(_sc_round_body, nchunks_per_tile, ngroups_per_tile,
                        zrows),
      out_type=jax.ShapeDtypeStruct((B, NC, h_alloc, D), f32),
      mesh=mesh,
      scratch_types=[
          pltpu.VMEM_SHARED((h_alloc, D), f32),
          pltpu.VMEM((GRP, CHUNK), jnp.int32),
          pltpu.VMEM((GRP, CHUNK), jnp.int32),
          pltpu.VMEM((2, CHUNK, D), f32),
          pltpu.SemaphoreType.DMA,
          pltpu.SemaphoreType.DMA,
          pltpu.SemaphoreType.DMA,
          pltpu.SemaphoreType.DMA,
      ],
      compiler_params=pltpu.CompilerParams(use_tc_tiling_on_sc=False),
  )

  wspec = [
      pl.BlockSpec((1, D), lambda b, i: (0, 0)),
      pl.BlockSpec((1, D), lambda b, i: (0, 0)),
      pl.BlockSpec((D, D), lambda b, i: (0, 0)),
  ]
  g_spec = pl.BlockSpec((1, BN, 2), lambda b, i: (b, i, 0))
  agg_spec = pl.BlockSpec(
      (1, 1, BN, D),
      lambda b, i, _n=nb_per_core: (b, i // _n, i % _n, 0))
  h_spec = pl.BlockSpec((1, BN, D), lambda b, i: (b, i, 0))

  h1 = pl.pallas_call(
      _tc_round1_body,
      grid=(B, nblk),
      in_specs=[g_spec] + wspec + [pl.BlockSpec((D, D), lambda b, i: (0, 0))],
      out_specs=h_spec,
      out_shape=jax.ShapeDtypeStruct((B, npad, D), f32),
  )(g, t1r, t4r, theta3, theta2)

  agg2 = round_kernel(h1, srcp, ldst, z2)

  h2 = pl.pallas_call(
      _tc_round_body,
      grid=(B, nblk),
      in_specs=[g_spec, agg_spec] + wspec
      + [pl.BlockSpec((D, D), lambda b, i: (0, 0))],
      out_specs=h_spec,
      out_shape=jax.ShapeDtypeStruct((B, npad, D), f32),
  )(g, agg2, t1r, t4r, theta3, theta2)

  agg3 = round_kernel(h2, srcp, ldst, z2)

  musum, muv = pl.pallas_call(
      functools.partial(_tc_final_round_body, nb_per_core, h_half),
      grid=(B, nblk),
      in_specs=[g_spec, agg_spec,
                pl.BlockSpec(memory_space=pltpu.SMEM)] + wspec,
      out_specs=[pl.BlockSpec((1, 1, D), lambda b, i: (b, 0, 0)),
                 pl.BlockSpec((1, 1, D), lambda b, i: (b, 0, 0))],
      out_shape=[jax.ShapeDtypeStruct((B, 1, D), f32),
                 jax.ShapeDtypeStruct((B, 1, D), f32)],
  )(g, agg3, vs, t1r, t4r, theta3)

  qfull = pl.pallas_call(
      _tc_q_body,
      grid=(B,),
      in_specs=[pl.BlockSpec((1, 1, D), lambda b: (b, 0, 0)),
                pl.BlockSpec((1, 1, D), lambda b: (b, 0, 0)),
                pl.BlockSpec((1, 2 * D), lambda b: (0, 0)),
                pl.BlockSpec((D, D), lambda b: (0, 0)),
                pl.BlockSpec((D, D), lambda b: (0, 0))],
      out_specs=pl.BlockSpec((1, 1, D), lambda b: (b, 0, 0)),
      out_shape=jax.ShapeDtypeStruct((B, 1, D), f32),
  )(musum, muv, t5r, theta6, theta7)

  return qfull[:, 0, :1]


# speedup vs baseline: 1.5420x; 1.5420x over previous
import functools

import jax
import jax.numpy as jnp
from jax import lax
from jax.experimental import pallas as pl
from jax.experimental.pallas import tpu as pltpu
from jax.experimental.pallas import tpu_sc as plsc

D = 64
NC = 2
NS = 16
BN = 1792

CHUNK = 128
GRP = 16
EG = CHUNK * GRP
NBUF = 3


def _sc_round_body(nchunks_per_tile, ngroups_per_tile, zrows, orows,
                   h_hbm, src_hbm, ldst_hbm, z2_hbm, agg_hbm,
                   acc, sidx, ldstv, rows,
                   gsem0, gsem1, gsem2, ssem0, ssem1, ssem2):
  gsems = [gsem0, gsem1, gsem2]
  ssems = [ssem0, ssem1, ssem2]
  c = lax.axis_index("c")
  s = lax.axis_index("s")
  for b in range(2):
    pltpu.sync_copy(z2_hbm, acc.at[pl.ds(s * zrows, zrows)])
    plsc.subcore_barrier()

    def grp_body(g, _):
      base = s * nchunks_per_tile + g * GRP
      pltpu.sync_copy(src_hbm.at[pl.ds(base, GRP)], sidx)
      pltpu.sync_copy(ldst_hbm.at[c, pl.ds(base, GRP)], ldstv)
      gds = [None] * GRP
      sds = [None] * GRP
      la = NBUF - 1
      for k in range(la):
        gds[k] = pltpu.async_copy(h_hbm.at[b].at[sidx.at[k]],
                                  rows.at[k % NBUF], gsems[k % NBUF])
      for j in range(GRP):
        k = j + la
        if k < GRP:
          if k >= NBUF:
            sds[k - NBUF].wait()
          gds[k] = pltpu.async_copy(h_hbm.at[b].at[sidx.at[k]],
                                    rows.at[k % NBUF], gsems[k % NBUF])
        gds[j].wait()
        sds[j] = pltpu.async_copy(rows.at[j % NBUF], acc.at[ldstv.at[j]],
                                  ssems[j % NBUF], add=True)
      for j in range(max(0, GRP - NBUF), GRP):
        sds[j].wait()
      return 0

    lax.fori_loop(0, ngroups_per_tile, grp_body, 0)
    plsc.subcore_barrier()
    pltpu.sync_copy(acc.at[pl.ds(s * orows, orows)],
                    agg_hbm.at[b, c, pl.ds(s * orows, orows)])
    if b == 0:
      plsc.subcore_barrier()


def _sc_deg_body(nchunks_per_tile, ngroups_per_tile, zrows,
                 w_hbm, ldst_hbm, z1_hbm, deg_hbm,
                 acc1, wv, ldstv, zb1):
  c = lax.axis_index("c")
  s = lax.axis_index("s")
  pltpu.sync_copy(z1_hbm, zb1)
  pltpu.sync_copy(zb1, acc1.at[pl.ds(s * zrows, zrows)])
  plsc.subcore_barrier()

  def grp_body(g, _):
    base = s * nchunks_per_tile + g * GRP
    pltpu.sync_copy(w_hbm.at[pl.ds(base, GRP)], wv)
    pltpu.sync_copy(ldst_hbm.at[c, pl.ds(base, GRP)], ldstv)
    for j in range(GRP):
      pltpu.sync_copy(wv.at[j], acc1.at[ldstv.at[j]], add=True)
    return 0

  lax.fori_loop(0, ngroups_per_tile, grp_body, 0)
  plsc.subcore_barrier()
  halloc = zrows * NS
  pltpu.sync_copy(acc1.at[pl.ds(s * zrows, zrows)], zb1)
  pltpu.sync_copy(zb1, deg_hbm.at[pl.ds(c * halloc + s * zrows, zrows)])


def _base_term(g_blk, t1, t4, t3):
  cvec = lax.dot_general(jnp.maximum(t4, 0.0), t3,
                         (((1,), (1,)), ((), ())))
  return g_blk[:, 0:1] * t1 + g_blk[:, 1:2] * cvec


def _tc_round1_body(g_ref, t1_ref, t4_ref, t3_ref, t2_ref, h_ref):
  base = _base_term(g_ref[0], t1_ref[...], t4_ref[...], t3_ref[...])
  mu = jnp.maximum(base, 0.0)
  h_ref[0] = lax.dot_general(mu, t2_ref[...], (((1,), (1,)), ((), ())))


def _tc_round_body(g_ref, agg_ref, t1_ref, t4_ref, t3_ref, t2_ref, h_ref):
  base = _base_term(g_ref[0], t1_ref[...], t4_ref[...], t3_ref[...])
  mu = jnp.maximum(base + agg_ref[0, 0], 0.0)
  h_ref[0] = lax.dot_general(mu, t2_ref[...], (((1,), (1,)), ((), ())))


def _tc_final_round_body(nb_per_core, h_half, nblk,
                         g_ref, agg_ref, vs_ref, t1_ref, t4_ref, t3_ref,
                         t5_ref, t6_ref, t7_ref, q_ref, ms_acc, mv_acc):
  b = pl.program_id(0)
  i = pl.program_id(1)
  base = _base_term(g_ref[0], t1_ref[...], t4_ref[...], t3_ref[...])
  mu = jnp.maximum(base + agg_ref[0, 0], 0.0)

  @pl.when(i == 0)
  def _():
    ms_acc[...] = jnp.zeros_like(ms_acc)
    mv_acc[...] = jnp.zeros_like(mv_acc)

  ms_acc[...] += jnp.sum(mu, axis=0)[None, :]
  r = vs_ref[b] - (i // nb_per_core) * h_half - (i % nb_per_core) * BN
  iota = lax.broadcasted_iota(jnp.int32, (BN, 1), 0)
  hit = jnp.where(iota == r, mu, 0.0)
  mv_acc[...] += jnp.sum(hit, axis=0)[None, :]

  @pl.when(i == nblk - 1)
  def _():
    ms = ms_acc[...]
    mv = mv_acc[...]
    h6 = jnp.maximum(
        lax.dot_general(ms, t6_ref[...], (((1,), (1,)), ((), ()))), 0.0)
    h7 = jnp.maximum(
        lax.dot_general(mv, t7_ref[...], (((1,), (1,)), ((), ()))), 0.0)
    t5 = t5_ref[...]
    q = jnp.sum(h6 * t5[:, :D]) + jnp.sum(h7 * t5[:, D:])
    q_ref[...] = jnp.zeros_like(q_ref) + q


@jax.jit
def kernel(xv, vs, edge_index, edge_weight, theta1, theta2, theta3, theta4,
           theta5, theta6, theta7):
  B, N = xv.shape
  E = edge_index.shape[1]

  h_half = ((N + 2 * BN - 1) // (2 * BN)) * BN
  npad = 2 * h_half
  nb_per_core = h_half // BN
  nblk = npad // BN
  ndump = 256
  h_alloc = h_half + ndump
  zrows = h_alloc // NS
  orows = h_half // NS
  ept = ((E + NS * EG - 1) // (NS * EG)) * EG
  e_pad = NS * ept
  nchunks_per_tile = ept // CHUNK
  ngroups_per_tile = ept // EG
  npad_edges = e_pad - E

  f32 = jnp.float32

  src = edge_index[0]
  dst = edge_index[1]
  pad_src = jnp.arange(npad_edges, dtype=jnp.int32) % N
  srcp = jnp.concatenate([src, pad_src]).reshape(e_pad // CHUNK, CHUNK)
  dstp = jnp.concatenate(
      [dst, jnp.full((npad_edges,), npad, jnp.int32)])
  lane = jnp.arange(e_pad, dtype=jnp.int32) % ndump
  dump = h_half + lane
  ldst0 = jnp.where(dstp < h_half, dstp, dump)
  ldst1 = jnp.where((dstp >= h_half) & (dstp < npad), dstp - h_half, dump)
  ldst = jnp.stack([ldst0, ldst1]).reshape(NC, e_pad // CHUNK, CHUNK)
  wp = jnp.concatenate([edge_weight, jnp.zeros((npad_edges,), f32)]
                       ).reshape(e_pad // CHUNK, CHUNK)
  xp = jnp.concatenate([xv, jnp.zeros((B, npad - N), f32)], axis=1)
  z1 = jnp.zeros((zrows,), f32)
  z2 = jnp.zeros((zrows, D), f32)
  t1r = theta1.reshape(1, D)
  t4r = theta4.reshape(1, D)
  t5r = theta5.reshape(1, 2 * D)

  mesh = plsc.VectorSubcoreMesh(core_axis_name="c", subcore_axis_name="s",
                                num_cores=NC, num_subcores=NS)

  deg_kernel = pl.kernel(
      functools.partial(_sc_deg_body, nchunks_per_tile, ngroups_per_tile,
                        zrows),
      out_type=jax.ShapeDtypeStruct((NC * h_alloc,), f32),
      mesh=mesh,
      scratch_types=[
          pltpu.VMEM_SHARED((h_alloc,), f32),
          pltpu.VMEM((GRP, CHUNK), f32),
          pltpu.VMEM((GRP, CHUNK), jnp.int32),
          pltpu.VMEM((zrows,), f32),
      ],
  )
  deg_sc = deg_kernel(wp, ldst, z1)
  deg_lin = deg_sc.reshape(NC, h_alloc)[:, :h_half].reshape(npad)

  g = jnp.stack([xp, jnp.broadcast_to(deg_lin[None, :], (B, npad))],
                axis=-1)

  round_kernel = pl.kernel(
      functools.partial(_sc_round_body, nchunks_per_tile, ngroups_per_tile,
                        zrows, orows),
      out_type=jax.ShapeDtypeStruct((B, NC, h_half, D), f32),
      mesh=mesh,
      scratch_types=[
          pltpu.VMEM_SHARED((h_alloc, D), f32),
          pltpu.VMEM((GRP, CHUNK), jnp.int32),
          pltpu.VMEM((GRP, CHUNK), jnp.int32),
          pltpu.VMEM((NBUF, CHUNK, D), f32),
          pltpu.SemaphoreType.DMA,
          pltpu.SemaphoreType.DMA,
          pltpu.SemaphoreType.DMA,
          pltpu.SemaphoreType.DMA,
          pltpu.SemaphoreType.DMA,
          pltpu.SemaphoreType.DMA,
      ],
      compiler_params=pltpu.CompilerParams(use_tc_tiling_on_sc=False),
  )

  wspec = [
      pl.BlockSpec((1, D), lambda b, i: (0, 0)),
      pl.BlockSpec((1, D), lambda b, i: (0, 0)),
      pl.BlockSpec((D, D), lambda b, i: (0, 0)),
  ]
  g_spec = pl.BlockSpec((1, BN, 2), lambda b, i: (b, i, 0))
  agg_spec = pl.BlockSpec(
      (1, 1, BN, D),
      lambda b, i, _n=nb_per_core: (b, i // _n, i % _n, 0))
  h_spec = pl.BlockSpec((1, BN, D), lambda b, i: (b, i, 0))

  h1 = pl.pallas_call(
      _tc_round1_body,
      grid=(B, nblk),
      in_specs=[g_spec] + wspec + [pl.BlockSpec((D, D), lambda b, i: (0, 0))],
      out_specs=h_spec,
      out_shape=jax.ShapeDtypeStruct((B, npad, D), f32),
  )(g, t1r, t4r, theta3, theta2)

  agg2 = round_kernel(h1, srcp, ldst, z2)

  h2 = pl.pallas_call(
      _tc_round_body,
      grid=(B, nblk),
      in_specs=[g_spec, agg_spec] + wspec
      + [pl.BlockSpec((D, D), lambda b, i: (0, 0))],
      out_specs=h_spec,
      out_shape=jax.ShapeDtypeStruct((B, npad, D), f32),
  )(g, agg2, t1r, t4r, theta3, theta2)

  agg3 = round_kernel(h2, srcp, ldst, z2)

  qfull = pl.pallas_call(
      functools.partial(_tc_final_round_body, nb_per_core, h_half, nblk),
      grid=(B, nblk),
      in_specs=[g_spec, agg_spec,
                pl.BlockSpec(memory_space=pltpu.SMEM)] + wspec
      + [pl.BlockSpec((1, 2 * D), lambda b, i: (0, 0)),
         pl.BlockSpec((D, D), lambda b, i: (0, 0)),
         pl.BlockSpec((D, D), lambda b, i: (0, 0))],
      out_specs=pl.BlockSpec((1, 1, D), lambda b, i: (b, 0, 0)),
      out_shape=jax.ShapeDtypeStruct((B, 1, D), f32),
      scratch_shapes=[pltpu.VMEM((1, D), f32), pltpu.VMEM((1, D), f32)],
  )(g, agg3, vs, t1r, t4r, theta3, t5r, theta6, theta7)

  return qfull[:, 0, :1]


# speedup vs baseline: 1.6759x; 1.0869x over previous
import functools

import jax
import jax.numpy as jnp
from jax import lax
from jax.experimental import pallas as pl
from jax.experimental.pallas import tpu as pltpu
from jax.experimental.pallas import tpu_sc as plsc

D = 64
NC = 2
NS = 16
BN = 1792

CHUNK = 128
GRP = 8
EG = CHUNK * GRP
NBUF = 3

PT_EDGES = 25600
REGION = 26624
REGION_GROUPS = REGION // EG
SLOT = 2 * REGION
NDUMP = 256


def _sc_part_body(h_half, srcf_hbm, dstf_hbm, pre_src_hbm, pre_dump_hbm,
                  psrc_hbm, pldst_hbm, cnt_hbm,
                  buf_s, buf_l, sidx, didx, cntv):
  c = lax.axis_index("c")
  s = lax.axis_index("s")
  w = c * NS + s
  pltpu.sync_copy(pre_src_hbm, buf_s)
  pltpu.sync_copy(pre_dump_hbm, buf_l)
  ebase = w * PT_EDGES

  def grp_body(g, carry):
    pltpu.sync_copy(srcf_hbm.at[pl.ds(ebase + g * EG, EG)], sidx)
    pltpu.sync_copy(dstf_hbm.at[pl.ds(ebase + g * EG, EG)], didx)

    def vreg_body(t, cr):
      o_a, o_b = cr
      s16 = sidx[pl.ds(t * 16, 16)]
      d16 = didx[pl.ds(t * 16, 16)]
      m_a = d16 < h_half
      n_a = plsc.all_reduce_population_count(m_a)[0]
      plsc.store_compressed(buf_s.at[pl.ds(o_a, 16)], s16, mask=m_a)
      plsc.store_compressed(buf_l.at[pl.ds(o_a, 16)], d16, mask=m_a)
      m_b = jnp.logical_not(m_a)
      plsc.store_compressed(buf_s.at[pl.ds(o_b, 16)], s16, mask=m_b)
      plsc.store_compressed(buf_l.at[pl.ds(o_b, 16)], d16 - h_half, mask=m_b)
      return (o_a + n_a, o_b + (16 - n_a))

    return lax.fori_loop(0, EG // 16, vreg_body, carry)

  c_a, o_b = lax.fori_loop(0, PT_EDGES // EG, grp_body,
                           (jnp.int32(0), jnp.int32(REGION)))
  lane16 = lax.iota(jnp.int32, 16)
  buf_s[pl.ds(c_a, 16)] = lane16
  buf_l[pl.ds(c_a, 16)] = h_half + lane16
  buf_s[pl.ds(o_b, 16)] = lane16
  buf_l[pl.ds(o_b, 16)] = h_half + lane16
  cntv[...] = jnp.zeros((16,), jnp.int32) + c_a
  pltpu.sync_copy(cntv, cnt_hbm.at[pl.ds(w * 16, 16)])
  pltpu.sync_copy(buf_s.at[pl.ds(0, SLOT)], psrc_hbm.at[pl.ds(w * SLOT, SLOT)])
  pltpu.sync_copy(buf_l.at[pl.ds(0, SLOT)], pldst_hbm.at[pl.ds(w * SLOT, SLOT)])


def _sc_round_body(zrows, orows,
                   h_hbm, psrc_hbm, pldst_hbm, cnt_hbm, z2_hbm, agg_hbm,
                   acc, sidx, ldstv, rows, cntv,
                   gsem0, gsem1, gsem2, ssem0, ssem1, ssem2):
  gsems = [gsem0, gsem1, gsem2]
  ssems = [ssem0, ssem1, ssem2]
  c = lax.axis_index("c")
  s = lax.axis_index("s")
  for b in range(2):
    pltpu.sync_copy(z2_hbm, acc.at[pl.ds(s * zrows, zrows)])
    plsc.subcore_barrier()

    for wslot in (2 * s, 2 * s + 1):
      pltpu.sync_copy(cnt_hbm.at[pl.ds(wslot * 16, 16)], cntv)
      c_a = cntv[...][0]
      glo = jnp.where(c == 0, 0, REGION_GROUPS)
      ghi = jnp.where(c == 0,
                      lax.shift_right_logical(c_a + (EG - 1), 10),
                      REGION_GROUPS
                      + lax.shift_right_logical(PT_EDGES - c_a + (EG - 1), 10))

      def grp_body(g, _):
        pltpu.sync_copy(psrc_hbm.at[wslot, pl.ds(g * GRP, GRP)], sidx)
        pltpu.sync_copy(pldst_hbm.at[wslot, pl.ds(g * GRP, GRP)], ldstv)
        gds = [None] * GRP
        sds = [None] * GRP
        la = NBUF - 1
        for k in range(la):
          gds[k] = pltpu.async_copy(h_hbm.at[b].at[sidx.at[k]],
                                    rows.at[k % NBUF], gsems[k % NBUF])
        for j in range(GRP):
          k = j + la
          if k < GRP:
            if k >= NBUF:
              sds[k - NBUF].wait()
            gds[k] = pltpu.async_copy(h_hbm.at[b].at[sidx.at[k]],
                                      rows.at[k % NBUF], gsems[k % NBUF])
          gds[j].wait()
          sds[j] = pltpu.async_copy(rows.at[j % NBUF], acc.at[ldstv.at[j]],
                                    ssems[j % NBUF], add=True)
        for j in range(max(0, GRP - NBUF), GRP):
          sds[j].wait()
        return 0

      lax.fori_loop(glo, ghi, grp_body, 0)
    plsc.subcore_barrier()
    pltpu.sync_copy(acc.at[pl.ds(s * orows, orows)],
                    agg_hbm.at[b, c, pl.ds(s * orows, orows)])
    if b == 0:
      plsc.subcore_barrier()


def _sc_deg_body(nchunks_per_tile, ngroups_per_tile, zrows,
                 w_hbm, ldst_hbm, z1_hbm, deg_hbm,
                 acc1, wv, ldstv, zb1):
  c = lax.axis_index("c")
  s = lax.axis_index("s")
  pltpu.sync_copy(z1_hbm, zb1)
  pltpu.sync_copy(zb1, acc1.at[pl.ds(s * zrows, zrows)])
  plsc.subcore_barrier()

  def grp_body(g, _):
    base = s * nchunks_per_tile + g * GRP
    pltpu.sync_copy(w_hbm.at[pl.ds(base, GRP)], wv)
    pltpu.sync_copy(ldst_hbm.at[c, pl.ds(base, GRP)], ldstv)
    for j in range(GRP):
      pltpu.sync_copy(wv.at[j], acc1.at[ldstv.at[j]], add=True)
    return 0

  lax.fori_loop(0, ngroups_per_tile, grp_body, 0)
  plsc.subcore_barrier()
  halloc = zrows * NS
  pltpu.sync_copy(acc1.at[pl.ds(s * zrows, zrows)], zb1)
  pltpu.sync_copy(zb1, deg_hbm.at[pl.ds(c * halloc + s * zrows, zrows)])


def _base_term(g_blk, t1, t4, t3):
  cvec = lax.dot_general(jnp.maximum(t4, 0.0), t3,
                         (((1,), (1,)), ((), ())), precision=lax.Precision.HIGHEST)
  return g_blk[:, 0:1] * t1 + g_blk[:, 1:2] * cvec


def _tc_round1_body(g_ref, t1_ref, t4_ref, t3_ref, mu_ref):
  base = _base_term(g_ref[0], t1_ref[...], t4_ref[...], t3_ref[...])
  mu_ref[0] = jnp.maximum(base, 0.0)


def _tc_round_body(g_ref, agg_ref, t1_ref, t4_ref, t3_ref, t2_ref, mu_ref):
  base = _base_term(g_ref[0], t1_ref[...], t4_ref[...], t3_ref[...])
  agg2 = lax.dot_general(agg_ref[0, 0], t2_ref[...], (((1,), (1,)), ((), ())), precision=lax.Precision.HIGHEST)
  mu_ref[0] = jnp.maximum(base + agg2, 0.0)


def _tc_final_round_body(nb_per_core, h_half, nblk,
                         g_ref, agg_ref, vs_ref, t1_ref, t4_ref, t3_ref,
                         t2_ref, t5_ref, t6_ref, t7_ref, q_ref, ms_acc,
                         mv_acc):
  b = pl.program_id(0)
  i = pl.program_id(1)
  base = _base_term(g_ref[0], t1_ref[...], t4_ref[...], t3_ref[...])
  agg2 = lax.dot_general(agg_ref[0, 0], t2_ref[...], (((1,), (1,)), ((), ())), precision=lax.Precision.HIGHEST)
  mu = jnp.maximum(base + agg2, 0.0)

  @pl.when(i == 0)
  def _():
    ms_acc[...] = jnp.zeros_like(ms_acc)
    mv_acc[...] = jnp.zeros_like(mv_acc)

  ms_acc[...] += jnp.sum(mu, axis=0)[None, :]
  r = vs_ref[b] - (i // nb_per_core) * h_half - (i % nb_per_core) * BN
  iota = lax.broadcasted_iota(jnp.int32, (BN, 1), 0)
  hit = jnp.where(iota == r, mu, 0.0)
  mv_acc[...] += jnp.sum(hit, axis=0)[None, :]

  @pl.when(i == nblk - 1)
  def _():
    ms = ms_acc[...]
    mv = mv_acc[...]
    h6 = jnp.maximum(
        lax.dot_general(ms, t6_ref[...], (((1,), (1,)), ((), ())), precision=lax.Precision.HIGHEST), 0.0)
    h7 = jnp.maximum(
        lax.dot_general(mv, t7_ref[...], (((1,), (1,)), ((), ())), precision=lax.Precision.HIGHEST), 0.0)
    t5 = t5_ref[...]
    q = jnp.sum(h6 * t5[:, :D]) + jnp.sum(h7 * t5[:, D:])
    q_ref[...] = jnp.zeros_like(q_ref) + q


@jax.jit
def kernel(xv, vs, edge_index, edge_weight, theta1, theta2, theta3, theta4,
           theta5, theta6, theta7):
  B, N = xv.shape
  E = edge_index.shape[1]

  h_half = ((N + 2 * BN - 1) // (2 * BN)) * BN
  npad = 2 * h_half
  nb_per_core = h_half // BN
  nblk = npad // BN
  ndump = 256
  h_alloc = h_half + ndump
  zrows = h_alloc // NS
  orows = h_half // NS
  assert PT_EDGES == ((E + NC * NS * EG - 1) // (NC * NS * EG)) * EG
  e_pad = NC * NS * PT_EDGES
  nchunks_per_tile = e_pad // (NS * CHUNK)
  ngroups_per_tile = nchunks_per_tile // GRP
  npad_edges = e_pad - E

  f32 = jnp.float32

  src = edge_index[0]
  dst = edge_index[1]
  arange_pad = jnp.arange(npad_edges, dtype=jnp.int32)
  pad_src = arange_pad % N
  pad_dst = npad + (arange_pad % (ndump - 16))
  srcf = jnp.concatenate([src, pad_src])
  dstf = jnp.concatenate([dst, pad_dst])
  lane = jnp.arange(e_pad, dtype=jnp.int32) % ndump
  dump = h_half + lane
  ldst0 = jnp.where(dstf < h_half, dstf, dump)
  ldst1 = jnp.where((dstf >= h_half) & (dstf < npad), dstf - h_half, dump)
  ldst = jnp.stack([ldst0, ldst1]).reshape(NC, e_pad // CHUNK, CHUNK)
  wp = jnp.concatenate([edge_weight, jnp.zeros((npad_edges,), f32)]
                       ).reshape(e_pad // CHUNK, CHUNK)
  arange_slot = jnp.arange(SLOT + 16, dtype=jnp.int32)
  pre_src = (arange_slot * 997) % N
  pre_dump = h_half + (arange_slot % ndump)
  xp = jnp.concatenate([xv, jnp.zeros((B, npad - N), f32)], axis=1)
  z1 = jnp.zeros((zrows,), f32)
  z2 = jnp.zeros((zrows, D), f32)
  t1r = theta1.reshape(1, D)
  t4r = theta4.reshape(1, D)
  t5r = theta5.reshape(1, 2 * D)

  mesh = plsc.VectorSubcoreMesh(core_axis_name="c", subcore_axis_name="s",
                                num_cores=NC, num_subcores=NS)

  deg_kernel = pl.kernel(
      functools.partial(_sc_deg_body, nchunks_per_tile, ngroups_per_tile,
                        zrows),
      out_type=jax.ShapeDtypeStruct((NC * h_alloc,), f32),
      mesh=mesh,
      scratch_types=[
          pltpu.VMEM_SHARED((h_alloc,), f32),
          pltpu.VMEM((GRP, CHUNK), f32),
          pltpu.VMEM((GRP, CHUNK), jnp.int32),
          pltpu.VMEM((zrows,), f32),
      ],
  )
  deg_sc = deg_kernel(wp, ldst, z1)
  deg_lin = deg_sc.reshape(NC, h_alloc)[:, :h_half].reshape(npad)

  g = jnp.stack([xp, jnp.broadcast_to(deg_lin[None, :], (B, npad))],
                axis=-1)

  part_kernel = pl.kernel(
      functools.partial(_sc_part_body, h_half),
      out_type=(jax.ShapeDtypeStruct((NC * NS * SLOT,), jnp.int32),
                jax.ShapeDtypeStruct((NC * NS * SLOT,), jnp.int32),
                jax.ShapeDtypeStruct((NC * NS * 16,), jnp.int32)),
      mesh=mesh,
      scratch_types=[
          pltpu.VMEM((SLOT + 16,), jnp.int32),
          pltpu.VMEM((SLOT + 16,), jnp.int32),
          pltpu.VMEM((EG,), jnp.int32),
          pltpu.VMEM((EG,), jnp.int32),
          pltpu.VMEM((16,), jnp.int32),
      ],
      compiler_params=pltpu.CompilerParams(use_tc_tiling_on_sc=False,
                                           needs_layout_passes=False),
  )
  psrc_f, pldst_f, cnts = part_kernel(srcf, dstf, pre_src, pre_dump)
  psrc = psrc_f.reshape(NC * NS, SLOT // CHUNK, CHUNK)
  pldst = pldst_f.reshape(NC * NS, SLOT // CHUNK, CHUNK)

  round_kernel = pl.kernel(
      functools.partial(_sc_round_body, zrows, orows),
      out_type=jax.ShapeDtypeStruct((B, NC, h_half, D), f32),
      mesh=mesh,
      scratch_types=[
          pltpu.VMEM_SHARED((h_alloc, D), f32),
          pltpu.VMEM((GRP, CHUNK), jnp.int32),
          pltpu.VMEM((GRP, CHUNK), jnp.int32),
          pltpu.VMEM((NBUF, CHUNK, D), f32),
          pltpu.VMEM((16,), jnp.int32),
          pltpu.SemaphoreType.DMA,
          pltpu.SemaphoreType.DMA,
          pltpu.SemaphoreType.DMA,
          pltpu.SemaphoreType.DMA,
          pltpu.SemaphoreType.DMA,
          pltpu.SemaphoreType.DMA,
      ],
      compiler_params=pltpu.CompilerParams(use_tc_tiling_on_sc=False),
  )

  wspec = [
      pl.BlockSpec((1, D), lambda b, i: (0, 0)),
      pl.BlockSpec((1, D), lambda b, i: (0, 0)),
      pl.BlockSpec((D, D), lambda b, i: (0, 0)),
  ]
  g_spec = pl.BlockSpec((1, BN, 2), lambda b, i: (b, i, 0))
  agg_spec = pl.BlockSpec(
      (1, 1, BN, D),
      lambda b, i, _n=nb_per_core: (b, i // _n, i % _n, 0))
  h_spec = pl.BlockSpec((1, BN, D), lambda b, i: (b, i, 0))

  h1 = pl.pallas_call(
      _tc_round1_body,
      grid=(B, nblk),
      in_specs=[g_spec] + wspec,
      out_specs=h_spec,
      out_shape=jax.ShapeDtypeStruct((B, npad, D), f32),
  )(g, t1r, t4r, theta3)

  agg2 = round_kernel(h1, psrc, pldst, cnts, z2)

  h2 = pl.pallas_call(
      _tc_round_body,
      grid=(B, nblk),
      in_specs=[g_spec, agg_spec] + wspec
      + [pl.BlockSpec((D, D), lambda b, i: (0, 0))],
      out_specs=h_spec,
      out_shape=jax.ShapeDtypeStruct((B, npad, D), f32),
  )(g, agg2, t1r, t4r, theta3, theta2)

  agg3 = round_kernel(h2, psrc, pldst, cnts, z2)

  qfull = pl.pallas_call(
      functools.partial(_tc_final_round_body, nb_per_core, h_half, nblk),
      grid=(B, nblk),
      in_specs=[g_spec, agg_spec,
                pl.BlockSpec(memory_space=pltpu.SMEM)] + wspec
      + [pl.BlockSpec((D, D), lambda b, i: (0, 0)),
         pl.BlockSpec((1, 2 * D), lambda b, i: (0, 0)),
         pl.BlockSpec((D, D), lambda b, i: (0, 0)),
         pl.BlockSpec((D, D), lambda b, i: (0, 0))],
      out_specs=pl.BlockSpec((1, 1, D), lambda b, i: (b, 0, 0)),
      out_shape=jax.ShapeDtypeStruct((B, 1, D), f32),
      scratch_shapes=[pltpu.VMEM((1, D), f32), pltpu.VMEM((1, D), f32)],
  )(g, agg3, vs, t1r, t4r, theta3, theta2, t5r, theta6, theta7)

  return qfull[:, 0, :1]
